# baseline (speedup 1.0000x reference)
import jax
import jax.numpy as jnp
from jax import lax
from jax.experimental import pallas as pl
from jax.experimental.pallas import tpu as pltpu
from jax.experimental.pallas import tpu_sc as plsc

_B = 16384
_K = 20
_D = 16
_L = 16
_NC = 2
_NS = 16
_NW = _NC * _NS
_BPW = _B // _NW
_CB = 128
_NCHUNK = _BPW // _CB
_NIW = 128
_NJ = _CB * _K // _NIW
_NROW = 1 + _K


def _sc_body(cw_hbm, pw_hbm, nw_hbm, in_hbm, out_hbm, sc_hbm,
             cidx_v, pidx_v, nidx_v, crow_v, prow_v, nrow_v, scr_v, sem):
  c = lax.axis_index("c")
  s = lax.axis_index("s")
  wid = s * _NC + c
  base = wid * _BPW

  @pl.loop(0, _NCHUNK)
  def _chunk(ci):
    b0 = base + ci * _CB
    pltpu.sync_copy(cw_hbm.at[pl.ds(b0, _CB)], cidx_v)
    pltpu.sync_copy(pw_hbm.at[pl.ds(b0, _CB)], pidx_v)
    pltpu.sync_copy(nw_hbm.at[pl.ds(b0 * _K, _CB * _K)], nidx_v)
    copies = [
        pltpu.async_copy(in_hbm.at[cidx_v], crow_v, sem),
        pltpu.async_copy(out_hbm.at[pidx_v], prow_v, sem),
    ]
    for j in range(_NJ):
      copies.append(pltpu.async_copy(
          out_hbm.at[nidx_v.at[pl.ds(j * _NIW, _NIW)]],
          nrow_v.at[pl.ds(j * _NIW, _NIW)], sem))
    for cp in copies:
      cp.wait()

    @pl.loop(0, _CB // _L)
    def _group(g):
      row0 = g * _L
      lane = lax.iota(jnp.int32, 16)
      rowi = row0 + lane
      rowk = rowi * _K
      cols = [jnp.full((16,), d, dtype=jnp.int32) for d in range(_D)]
      ccols = [plsc.load_gather(crow_v, [rowi, cols[d]]) for d in range(_D)]
      acc = ccols[0] * plsc.load_gather(prow_v, [rowi, cols[0]])
      for d in range(1, _D):
        acc = acc + ccols[d] * plsc.load_gather(prow_v, [rowi, cols[d]])
      scr_v[0, pl.ds(row0, _L)] = acc
      for k in range(_K):
        ri = rowk + k
        acc = ccols[0] * plsc.load_gather(nrow_v, [ri, cols[0]])
        for d in range(1, _D):
          acc = acc + ccols[d] * plsc.load_gather(nrow_v, [ri, cols[d]])
        scr_v[1 + k, pl.ds(row0, _L)] = -acc

    pltpu.sync_copy(scr_v, sc_hbm.at[:, pl.ds(b0, _CB)])


_sc_gather = pl.kernel(
    _sc_body,
    out_type=jax.ShapeDtypeStruct((_NROW, _B), jnp.float32),
    mesh=plsc.VectorSubcoreMesh(core_axis_name="c", subcore_axis_name="s"),
    compiler_params=pltpu.CompilerParams(
        needs_layout_passes=False, use_tc_tiling_on_sc=False),
    scratch_types=[
        pltpu.VMEM((_CB,), jnp.int32),
        pltpu.VMEM((_CB,), jnp.int32),
        pltpu.VMEM((_CB * _K,), jnp.int32),
        pltpu.VMEM((_CB, _D), jnp.float32),
        pltpu.VMEM((_CB, _D), jnp.float32),
        pltpu.VMEM((_CB * _K, _D), jnp.float32),
        pltpu.VMEM((_NROW, _CB), jnp.float32),
        pltpu.SemaphoreType.DMA,
    ],
)


def _loss_body(s_ref, o_ref):
  x = s_ref[...]
  o_ref[0, 0] = -jnp.sum(jax.nn.log_sigmoid(x)) / _B


_loss_call = pl.pallas_call(
    _loss_body,
    out_shape=jax.ShapeDtypeStruct((1, 1), jnp.float32),
    out_specs=pl.BlockSpec(memory_space=pltpu.SMEM),
)


def kernel(center_words, pos_context_words, neg_context_words, in_embed, out_embed):
  cw = center_words.astype(jnp.int32)
  pw = pos_context_words.astype(jnp.int32)
  nw = neg_context_words.astype(jnp.int32).reshape(_B * _K)
  scores = _sc_gather(cw, pw, nw, in_embed, out_embed)
  return _loss_call(scores).reshape(())
